# baseline (device time: 19539 ns/iter reference)
import jax
import jax.numpy as jnp
from jax import lax
from jax.experimental import pallas as pl
from jax.experimental.pallas import tpu as pltpu

N_DEV = 4
N_LOCAL_EXPERTS = 2


def kernel(x, router_W, route_idx, expert_W):
    n_tok, d_model = x.shape
    _, _, d_out = expert_W.shape

    def body(x_ref, idx_ref, ew_ref, out_ref, comm_ref, send_sems, recv_sems):
        my_pos = lax.axis_index("i")
        left = lax.rem(my_pos + N_DEV - 1, N_DEV)
        right = lax.rem(my_pos + 1, N_DEV)

        barrier_sem = pltpu.get_barrier_semaphore()
        for nbr in (left, right):
            pl.semaphore_signal(
                barrier_sem, inc=1,
                device_id=(nbr,), device_id_type=pl.DeviceIdType.MESH,
            )
        pl.semaphore_wait(barrier_sem, 2)

        idx = idx_ref[:, :]
        xb = x_ref[:, :].astype(jnp.bfloat16)
        acc = jnp.zeros((n_tok, d_out), jnp.float32)
        for k in range(N_LOCAL_EXPERTS):
            e = my_pos * N_LOCAL_EXPERTS + k
            xm = jnp.where(idx == e, xb, jnp.bfloat16(0))
            acc = acc + jnp.dot(
                xm, ew_ref[k].astype(jnp.bfloat16),
                preferred_element_type=jnp.float32,
            )
        out_ref[:, :] = acc
        comm_ref[0] = acc

        for h in range(N_DEV - 1):
            rdma = pltpu.make_async_remote_copy(
                src_ref=comm_ref.at[h],
                dst_ref=comm_ref.at[h + 1],
                send_sem=send_sems.at[h],
                recv_sem=recv_sems.at[h],
                device_id=(right,),
                device_id_type=pl.DeviceIdType.MESH,
            )
            rdma.start()
            rdma.wait()
            out_ref[:, :] += comm_ref[h + 1]

    return pl.pallas_call(
        body,
        out_shape=jax.ShapeDtypeStruct((n_tok, d_out), jnp.float32),
        in_specs=[pl.BlockSpec(memory_space=pltpu.VMEM)] * 3,
        out_specs=pl.BlockSpec(memory_space=pltpu.VMEM),
        scratch_shapes=[
            pltpu.VMEM((N_DEV, n_tok, d_out), jnp.float32),
            pltpu.SemaphoreType.DMA((N_DEV - 1,)),
            pltpu.SemaphoreType.DMA((N_DEV - 1,)),
        ],
        compiler_params=pltpu.CompilerParams(collective_id=0),
    )(x, route_idx, expert_W)


# device time: 10627 ns/iter; 1.8386x vs baseline; 1.8386x over previous
import jax
import jax.numpy as jnp
from jax import lax
from jax.experimental import pallas as pl
from jax.experimental.pallas import tpu as pltpu

N_DEV = 4
N_LOCAL_EXPERTS = 2


def kernel(x, router_W, route_idx, expert_W):
    n_tok, d_model = x.shape
    _, _, d_out = expert_W.shape

    def body(x_ref, idx_ref, ew_ref, out_ref, comm_ref, send_sems, recv_sems):
        my_pos = lax.axis_index("i")

        barrier_sem = pltpu.get_barrier_semaphore()
        for o in range(1, N_DEV):
            peer = lax.rem(my_pos + o, N_DEV)
            pl.semaphore_signal(
                barrier_sem, inc=1,
                device_id=(peer,), device_id_type=pl.DeviceIdType.MESH,
            )
        pl.semaphore_wait(barrier_sem, N_DEV - 1)

        idx = idx_ref[:, :]
        xb = x_ref[:, :].astype(jnp.bfloat16)
        acc = jnp.zeros((n_tok, d_out), jnp.float32)
        for k in range(N_LOCAL_EXPERTS):
            e = my_pos * N_LOCAL_EXPERTS + k
            xm = jnp.where(idx == e, xb, jnp.bfloat16(0))
            acc = acc + jnp.dot(
                xm, ew_ref[k].astype(jnp.bfloat16),
                preferred_element_type=jnp.float32,
            )
        comm_ref[0] = acc.astype(jnp.bfloat16)

        sends = []
        for o in (2, 1, 3):
            peer = lax.rem(my_pos + o, N_DEV)
            rdma = pltpu.make_async_remote_copy(
                src_ref=comm_ref.at[0],
                dst_ref=comm_ref.at[o],
                send_sem=send_sems.at[o],
                recv_sem=recv_sems.at[o],
                device_id=(peer,),
                device_id_type=pl.DeviceIdType.MESH,
            )
            rdma.start()
            sends.append(rdma)

        out_ref[:, :] = acc

        for j in (1, 3, 2):
            recv = pltpu.make_async_remote_copy(
                src_ref=comm_ref.at[0],
                dst_ref=comm_ref.at[j],
                send_sem=send_sems.at[0],
                recv_sem=recv_sems.at[j],
                device_id=(my_pos,),
                device_id_type=pl.DeviceIdType.MESH,
            )
            recv.wait_recv()

        out_ref[:, :] += (
            comm_ref[1] + comm_ref[2] + comm_ref[3]
        ).astype(jnp.float32)

        for r in sends:
            r.wait_send()

    return pl.pallas_call(
        body,
        out_shape=jax.ShapeDtypeStruct((n_tok, d_out), jnp.float32),
        in_specs=[pl.BlockSpec(memory_space=pltpu.VMEM)] * 3,
        out_specs=pl.BlockSpec(memory_space=pltpu.VMEM),
        scratch_shapes=[
            pltpu.VMEM((N_DEV, n_tok, d_out), jnp.bfloat16),
            pltpu.SemaphoreType.DMA((N_DEV,)),
            pltpu.SemaphoreType.DMA((N_DEV,)),
        ],
        compiler_params=pltpu.CompilerParams(collective_id=0),
    )(x, route_idx, expert_W)


# device time: 10314 ns/iter; 1.8944x vs baseline; 1.0303x over previous
import jax
import jax.numpy as jnp
from jax import lax
from jax.experimental import pallas as pl
from jax.experimental.pallas import tpu as pltpu

N_DEV = 4
N_LOCAL_EXPERTS = 2


def kernel(x, router_W, route_idx, expert_W):
    n_tok, d_model = x.shape
    _, _, d_out = expert_W.shape

    def body(x_ref, idx_ref, ew_ref, out_ref, comm_ref, send_sems, recv_sems):
        my_pos = lax.axis_index("i")

        barrier_sem = pltpu.get_barrier_semaphore()
        for o in range(1, N_DEV):
            peer = lax.rem(my_pos + o, N_DEV)
            pl.semaphore_signal(
                barrier_sem, inc=1,
                device_id=(peer,), device_id_type=pl.DeviceIdType.MESH,
            )

        idx = idx_ref[:, :]
        xb = x_ref[:, :].astype(jnp.bfloat16)
        acc = jnp.zeros((n_tok, d_out), jnp.float32)
        for k in range(N_LOCAL_EXPERTS):
            e = my_pos * N_LOCAL_EXPERTS + k
            xm = jnp.where(idx == e, xb, jnp.bfloat16(0))
            acc = acc + jnp.dot(
                xm, ew_ref[k].astype(jnp.bfloat16),
                preferred_element_type=jnp.float32,
            )
        comm_ref[0] = acc.astype(jnp.bfloat16)

        pl.semaphore_wait(barrier_sem, N_DEV - 1)

        sends = []
        for o in (2, 1, 3):
            peer = lax.rem(my_pos + o, N_DEV)
            rdma = pltpu.make_async_remote_copy(
                src_ref=comm_ref.at[0],
                dst_ref=comm_ref.at[o],
                send_sem=send_sems.at[o],
                recv_sem=recv_sems.at[o],
                device_id=(peer,),
                device_id_type=pl.DeviceIdType.MESH,
            )
            rdma.start()
            sends.append(rdma)

        for j in (1, 3, 2):
            recv = pltpu.make_async_remote_copy(
                src_ref=comm_ref.at[0],
                dst_ref=comm_ref.at[j],
                send_sem=send_sems.at[0],
                recv_sem=recv_sems.at[j],
                device_id=(my_pos,),
                device_id_type=pl.DeviceIdType.MESH,
            )
            recv.wait_recv()

        out_ref[:, :] = acc + (
            comm_ref[1] + comm_ref[2] + comm_ref[3]
        ).astype(jnp.float32)

        for r in sends:
            r.wait_send()

    return pl.pallas_call(
        body,
        out_shape=jax.ShapeDtypeStruct((n_tok, d_out), jnp.float32),
        in_specs=[pl.BlockSpec(memory_space=pltpu.VMEM)] * 3,
        out_specs=pl.BlockSpec(memory_space=pltpu.VMEM),
        scratch_shapes=[
            pltpu.VMEM((N_DEV, n_tok, d_out), jnp.bfloat16),
            pltpu.SemaphoreType.DMA((N_DEV,)),
            pltpu.SemaphoreType.DMA((N_DEV,)),
        ],
        compiler_params=pltpu.CompilerParams(collective_id=0),
    )(x, route_idx, expert_W)


# device time: 10143 ns/iter; 1.9264x vs baseline; 1.0169x over previous
import jax
import jax.numpy as jnp
from jax import lax
from jax.experimental import pallas as pl
from jax.experimental.pallas import tpu as pltpu

N_DEV = 4
N_LOCAL_EXPERTS = 2
CAP = 128


def kernel(x, router_W, route_idx, expert_W):
    n_tok, d_model = x.shape
    _, _, d_out = expert_W.shape

    def body(x_ref, idx_ref, ew_ref, out_ref, comm_ref, send_sems, recv_sems):
        my_pos = lax.axis_index("i")

        barrier_sem = pltpu.get_barrier_semaphore()
        for o in range(1, N_DEV):
            peer = lax.rem(my_pos + o, N_DEV)
            pl.semaphore_signal(
                barrier_sem, inc=1,
                device_id=(peer,), device_id_type=pl.DeviceIdType.MESH,
            )

        idx = idx_ref[:, :]
        owner = lax.div(idx, N_LOCAL_EXPERTS)
        row_i = lax.broadcasted_iota(jnp.int32, (CAP, 1), 0)
        tri = (
            lax.broadcasted_iota(jnp.int32, (n_tok, n_tok), 0)
            >= lax.broadcasted_iota(jnp.int32, (n_tok, n_tok), 1)
        ).astype(jnp.bfloat16)

        def gather_mat(p):
            mask = (owner == p)
            rank = jnp.dot(
                tri, mask.astype(jnp.bfloat16),
                preferred_element_type=jnp.float32,
            ).astype(jnp.int32) - 1
            rank_t = rank.reshape(1, n_tok)
            mask_t = mask.reshape(1, n_tok)
            return ((row_i == rank_t) & mask_t).astype(jnp.bfloat16)

        g_me = gather_mat(my_pos)
        xb = x_ref[:, :].astype(jnp.bfloat16)
        cx = jnp.dot(g_me, xb, preferred_element_type=jnp.float32
                     ).astype(jnp.bfloat16)
        f1 = jnp.dot(
            g_me, (idx == (my_pos * N_LOCAL_EXPERTS + 1)).astype(jnp.bfloat16),
            preferred_element_type=jnp.float32,
        ) > 0.5
        cacc = jnp.dot(
            jnp.where(f1, jnp.bfloat16(0), cx),
            ew_ref[0].astype(jnp.bfloat16),
            preferred_element_type=jnp.float32,
        ) + jnp.dot(
            jnp.where(f1, cx, jnp.bfloat16(0)),
            ew_ref[1].astype(jnp.bfloat16),
            preferred_element_type=jnp.float32,
        )
        comm_ref[0] = cacc.astype(jnp.bfloat16)

        pl.semaphore_wait(barrier_sem, N_DEV - 1)

        sends = []
        for o in (2, 1, 3):
            peer = lax.rem(my_pos + o, N_DEV)
            rdma = pltpu.make_async_remote_copy(
                src_ref=comm_ref.at[0],
                dst_ref=comm_ref.at[o],
                send_sem=send_sems.at[o],
                recv_sem=recv_sems.at[o],
                device_id=(peer,),
                device_id_type=pl.DeviceIdType.MESH,
            )
            rdma.start()
            sends.append(rdma)

        out = lax.dot_general(
            g_me, comm_ref[0],
            (((0,), (0,)), ((), ())),
            preferred_element_type=jnp.float32,
        )

        for j in (1, 3, 2):
            recv = pltpu.make_async_remote_copy(
                src_ref=comm_ref.at[0],
                dst_ref=comm_ref.at[j],
                send_sem=send_sems.at[0],
                recv_sem=recv_sems.at[j],
                device_id=(my_pos,),
                device_id_type=pl.DeviceIdType.MESH,
            )
            recv.wait_recv()
            src = lax.rem(my_pos - j + N_DEV, N_DEV)
            out = out + lax.dot_general(
                gather_mat(src), comm_ref[j],
                (((0,), (0,)), ((), ())),
                preferred_element_type=jnp.float32,
            )

        out_ref[:, :] = out

        for r in sends:
            r.wait_send()

    return pl.pallas_call(
        body,
        out_shape=jax.ShapeDtypeStruct((n_tok, d_out), jnp.float32),
        in_specs=[pl.BlockSpec(memory_space=pltpu.VMEM)] * 3,
        out_specs=pl.BlockSpec(memory_space=pltpu.VMEM),
        scratch_shapes=[
            pltpu.VMEM((N_DEV, CAP, d_out), jnp.bfloat16),
            pltpu.SemaphoreType.DMA((N_DEV,)),
            pltpu.SemaphoreType.DMA((N_DEV,)),
        ],
        compiler_params=pltpu.CompilerParams(collective_id=0),
    )(x, route_idx, expert_W)


# device time: 8963 ns/iter; 2.1800x vs baseline; 1.1317x over previous
import jax
import jax.numpy as jnp
from jax import lax
from jax.experimental import pallas as pl
from jax.experimental.pallas import tpu as pltpu

N_DEV = 4
N_LOCAL_EXPERTS = 2
CAP = 96
N_CHUNK = 2


def kernel(x, router_W, route_idx, expert_W):
    n_tok, d_model = x.shape
    _, _, d_out = expert_W.shape
    d_half = d_out // N_CHUNK

    def body(x_ref, idx_ref, ew_ref, out_ref, comm_ref, send_sems, recv_sems):
        my_pos = lax.axis_index("i")

        barrier_sem = pltpu.get_barrier_semaphore()
        for o in range(1, N_DEV):
            peer = lax.rem(my_pos + o, N_DEV)
            pl.semaphore_signal(
                barrier_sem, inc=1,
                device_id=(peer,), device_id_type=pl.DeviceIdType.MESH,
            )

        idx = idx_ref[:, :]
        owner = lax.div(idx, N_LOCAL_EXPERTS)
        row_i = lax.broadcasted_iota(jnp.int32, (CAP, 1), 0)
        tri = (
            lax.broadcasted_iota(jnp.int32, (n_tok, n_tok), 0)
            >= lax.broadcasted_iota(jnp.int32, (n_tok, n_tok), 1)
        ).astype(jnp.bfloat16)

        def gather_mat(p):
            mask = (owner == p)
            rank = jnp.dot(
                tri, mask.astype(jnp.bfloat16),
                preferred_element_type=jnp.float32,
            ).astype(jnp.int32) - 1
            rank_t = rank.reshape(1, n_tok)
            mask_t = mask.reshape(1, n_tok)
            return ((row_i == rank_t) & mask_t).astype(jnp.bfloat16)

        g_me = gather_mat(my_pos)
        xb = x_ref[:, :].astype(jnp.bfloat16)
        cx = jnp.dot(g_me, xb, preferred_element_type=jnp.float32
                     ).astype(jnp.bfloat16)
        f1 = jnp.dot(
            g_me, (idx == (my_pos * N_LOCAL_EXPERTS + 1)).astype(jnp.bfloat16),
            preferred_element_type=jnp.float32,
        ) > 0.5
        xm0 = jnp.where(f1, jnp.bfloat16(0), cx)
        xm1 = jnp.where(f1, cx, jnp.bfloat16(0))
        w0 = ew_ref[0].astype(jnp.bfloat16)
        w1 = ew_ref[1].astype(jnp.bfloat16)

        def chunk(c):
            lo, hi = c * d_half, (c + 1) * d_half
            return (
                jnp.dot(xm0, w0[:, lo:hi], preferred_element_type=jnp.float32)
                + jnp.dot(xm1, w1[:, lo:hi], preferred_element_type=jnp.float32)
            ).astype(jnp.bfloat16)

        comm_ref[0] = chunk(0)

        pl.semaphore_wait(barrier_sem, N_DEV - 1)

        def start_sends(c):
            rs = []
            for o in (2, 1, 3):
                peer = lax.rem(my_pos + o, N_DEV)
                slot = N_CHUNK * o + c
                rdma = pltpu.make_async_remote_copy(
                    src_ref=comm_ref.at[c],
                    dst_ref=comm_ref.at[slot],
                    send_sem=send_sems.at[slot],
                    recv_sem=recv_sems.at[slot],
                    device_id=(peer,),
                    device_id_type=pl.DeviceIdType.MESH,
                )
                rdma.start()
                rs.append(rdma)
            return rs

        sends = start_sends(0)
        comm_ref[1] = chunk(1)
        sends += start_sends(1)

        out_halves = [
            lax.dot_general(
                g_me, comm_ref[c],
                (((0,), (0,)), ((), ())),
                preferred_element_type=jnp.float32,
            )
            for c in range(N_CHUNK)
        ]
        g_peer = {
            j: gather_mat(lax.rem(my_pos - j + N_DEV, N_DEV))
            for j in (1, 2, 3)
        }

        for c in range(N_CHUNK):
            for j in (1, 3, 2):
                slot = N_CHUNK * j + c
                recv = pltpu.make_async_remote_copy(
                    src_ref=comm_ref.at[c],
                    dst_ref=comm_ref.at[slot],
                    send_sem=send_sems.at[0],
                    recv_sem=recv_sems.at[slot],
                    device_id=(my_pos,),
                    device_id_type=pl.DeviceIdType.MESH,
                )
                recv.wait_recv()
                out_halves[c] = out_halves[c] + lax.dot_general(
                    g_peer[j], comm_ref[slot],
                    (((0,), (0,)), ((), ())),
                    preferred_element_type=jnp.float32,
                )
            out_ref[:, c * d_half:(c + 1) * d_half] = out_halves[c]

        for r in sends:
            r.wait_send()

    return pl.pallas_call(
        body,
        out_shape=jax.ShapeDtypeStruct((n_tok, d_out), jnp.float32),
        in_specs=[pl.BlockSpec(memory_space=pltpu.VMEM)] * 3,
        out_specs=pl.BlockSpec(memory_space=pltpu.VMEM),
        scratch_shapes=[
            pltpu.VMEM((N_CHUNK * N_DEV, CAP, d_half), jnp.bfloat16),
            pltpu.SemaphoreType.DMA((N_CHUNK * N_DEV,)),
            pltpu.SemaphoreType.DMA((N_CHUNK * N_DEV,)),
        ],
        compiler_params=pltpu.CompilerParams(collective_id=0),
    )(x, route_idx, expert_W)
